# baseline (speedup 1.0000x reference)
import functools

import jax
import jax.numpy as jnp
from jax import lax
from jax.experimental import pallas as pl
from jax.experimental.pallas import tpu as pltpu
from jax.experimental.pallas import tpu_sc as plsc

N_ROWS = 2_000_000
L = 16
NC, NS = 2, 16
NW = NC * NS
VECS = N_ROWS // L
CHUNK_VECS = 500
CHUNK_ROWS = CHUNK_VECS * L
N_CHUNKS = VECS // CHUNK_VECS
BASE_CHUNKS = N_CHUNKS // NW
EXTRA = N_CHUNKS - BASE_CHUNKS * NW

_mesh = plsc.VectorSubcoreMesh(core_axis_name="c", subcore_axis_name="s")


@functools.partial(
    pl.kernel,
    mesh=_mesh,
    compiler_params=pltpu.CompilerParams(
        needs_layout_passes=False, use_tc_tiling_on_sc=False),
    out_type=jax.ShapeDtypeStruct((NW, L), jnp.float32),
    scratch_types=[
        pltpu.VMEM((2, CHUNK_ROWS), jnp.float32),
        pltpu.VMEM((2, CHUNK_ROWS), jnp.float32),
        pltpu.VMEM((2, CHUNK_ROWS), jnp.int32),
        pltpu.VMEM((L,), jnp.float32),
        pltpu.SemaphoreType.DMA,
        pltpu.SemaphoreType.DMA,
    ],
)
def _count_matches(logits_hbm, tgt_hbm, partials_hbm,
                   buf0, buf1, buf_t, acc_v, sem0, sem1):
    cid = lax.axis_index("c")
    sid = lax.axis_index("s")
    wid = sid * NC + cid
    first = wid * BASE_CHUNKS + jnp.minimum(wid, EXTRA)
    has_extra = wid < EXTRA

    sems = (sem0, sem1)

    def dma_triple(ci, slot):
        r0 = (first + ci) * CHUNK_ROWS
        sem = sems[slot]
        return (
            pltpu.make_async_copy(
                logits_hbm.at[0, pl.ds(r0, CHUNK_ROWS)], buf0.at[slot], sem),
            pltpu.make_async_copy(
                logits_hbm.at[1, pl.ds(r0, CHUNK_ROWS)], buf1.at[slot], sem),
            pltpu.make_async_copy(
                tgt_hbm.at[pl.ds(r0, CHUNK_ROWS)], buf_t.at[slot], sem),
        )

    def fire(ci, slot):
        for cp in dma_triple(ci, slot):
            cp.start()

    def drain(ci, slot):
        for cp in dma_triple(ci, slot):
            cp.wait()

    def compute(slot, acc):
        def compute_vec(j, acc):
            off = j * L
            o0 = buf0[slot, pl.ds(off, L)]
            o1 = buf1[slot, pl.ds(off, L)]
            t = buf_t[slot, pl.ds(off, L)]
            m = (o1 > o0) == (t > 0)
            return acc + jnp.where(m, 1.0, 0.0).astype(jnp.float32)
        return lax.fori_loop(0, CHUNK_VECS, compute_vec, acc, unroll=20)

    acc = jnp.zeros((L,), jnp.float32)
    fire(0, 0)
    for ci in range(BASE_CHUNKS):
        slot = ci % 2
        if ci + 1 < BASE_CHUNKS:
            fire(ci + 1, 1 - slot)
        else:
            @pl.when(has_extra)
            def _():
                fire(BASE_CHUNKS, 1 - slot)
        drain(ci, slot)
        acc = compute(slot, acc)

    tail_slot = BASE_CHUNKS % 2

    @pl.when(has_extra)
    def _():
        drain(BASE_CHUNKS, tail_slot)

    tail_acc = compute(tail_slot, acc)
    acc = jnp.where(has_extra, tail_acc, acc)

    acc_v[...] = acc
    pltpu.sync_copy(acc_v, partials_hbm.at[wid])


def kernel(output, target, segments):
    del segments
    partials = _count_matches(output.T, target)
    return jnp.sum(partials) / jnp.float32(N_ROWS)


# speedup vs baseline: 1.0089x; 1.0089x over previous
import functools

import jax
import jax.numpy as jnp
from jax import lax
from jax.experimental import pallas as pl
from jax.experimental.pallas import tpu as pltpu
from jax.experimental.pallas import tpu_sc as plsc

N_ROWS = 2_000_000
L = 16
NC, NS = 2, 16
NW = NC * NS
VECS = N_ROWS // L
CHUNK_VECS = 500
CHUNK_ROWS = CHUNK_VECS * L
N_CHUNKS = VECS // CHUNK_VECS
BASE_CHUNKS = N_CHUNKS // NW
EXTRA = N_CHUNKS - BASE_CHUNKS * NW

_mesh = plsc.VectorSubcoreMesh(core_axis_name="c", subcore_axis_name="s")


@functools.partial(
    pl.kernel,
    mesh=_mesh,
    compiler_params=pltpu.CompilerParams(
        needs_layout_passes=False, use_tc_tiling_on_sc=False),
    out_type=jax.ShapeDtypeStruct((NW, L), jnp.float32),
    scratch_types=[
        pltpu.VMEM((2, CHUNK_ROWS), jnp.float32),
        pltpu.VMEM((2, CHUNK_ROWS), jnp.float32),
        pltpu.VMEM((2, CHUNK_ROWS), jnp.int32),
        pltpu.VMEM((L,), jnp.float32),
        pltpu.SemaphoreType.DMA,
        pltpu.SemaphoreType.DMA,
    ],
)
def _count_matches(logits_hbm, tgt_hbm, partials_hbm,
                   buf0, buf1, buf_t, acc_v, sem0, sem1):
    cid = lax.axis_index("c")
    sid = lax.axis_index("s")
    wid = sid * NC + cid
    first = wid * BASE_CHUNKS + jnp.minimum(wid, EXTRA)
    has_extra = wid < EXTRA

    sems = (sem0, sem1)

    def dma_triple(ci, slot):
        r0 = (first + ci) * CHUNK_ROWS
        sem = sems[slot]
        return (
            pltpu.make_async_copy(
                logits_hbm.at[0, pl.ds(r0, CHUNK_ROWS)], buf0.at[slot], sem),
            pltpu.make_async_copy(
                logits_hbm.at[1, pl.ds(r0, CHUNK_ROWS)], buf1.at[slot], sem),
            pltpu.make_async_copy(
                tgt_hbm.at[pl.ds(r0, CHUNK_ROWS)], buf_t.at[slot], sem),
        )

    def fire(ci, slot):
        for cp in dma_triple(ci, slot):
            cp.start()

    def drain(ci, slot):
        for cp in dma_triple(ci, slot):
            cp.wait()

    def compute(slot, acc):
        def compute_vec(j, acc):
            off = j * L
            o0 = buf0[slot, pl.ds(off, L)]
            o1 = buf1[slot, pl.ds(off, L)]
            t = buf_t[slot, pl.ds(off, L)]
            m = (o1 > o0) == (t > 0)
            return acc + jnp.where(m, 1.0, 0.0).astype(jnp.float32)
        return lax.fori_loop(0, CHUNK_VECS, compute_vec, acc, unroll=8)

    acc = jnp.zeros((L,), jnp.float32)
    fire(0, 0)
    for ci in range(BASE_CHUNKS):
        slot = ci % 2
        if ci + 1 < BASE_CHUNKS:
            fire(ci + 1, 1 - slot)
        else:
            @pl.when(has_extra)
            def _():
                fire(BASE_CHUNKS, 1 - slot)
        drain(ci, slot)
        acc = compute(slot, acc)

    tail_slot = BASE_CHUNKS % 2

    @pl.when(has_extra)
    def _():
        drain(BASE_CHUNKS, tail_slot)

    tail_acc = compute(tail_slot, acc)
    acc = jnp.where(has_extra, tail_acc, acc)

    acc_v[...] = acc
    pltpu.sync_copy(acc_v, partials_hbm.at[wid])


def kernel(output, target, segments):
    del segments
    partials = _count_matches(output.T, target)
    return jnp.sum(partials) / jnp.float32(N_ROWS)
